# baseline (speedup 1.0000x reference)
import functools

import jax
import jax.numpy as jnp
from jax import lax
from jax.experimental import pallas as pl
from jax.experimental.pallas import tpu as pltpu
from jax.experimental.pallas import tpu_sc as plsc

N = 10000
E = 320000
D = 128
NC, NS = 2, 16
NT = NC * NS
CH = 64
EPT = 10240
EP = EPT * NT
NCHUNK = EPT // CH
NPT = 384
NP = NPT * NT
NA = 10240
STRIDE = NA // NS

f32 = jnp.float32
i32 = jnp.int32


def _mesh():
    return plsc.VectorSubcoreMesh(
        core_axis_name="c", subcore_axis_name="s", num_cores=NC, num_subcores=NS
    )


def _sc_init(nd, ed, ohp, table, zbig, h0_out, cnt_out, acc_cnt, ndb, edb,
             rb0, oh128, oh0, oh1, sem, semo0, semo1):
    c = lax.axis_index("c")
    s = lax.axis_index("s")
    w = s * NC + c
    pltpu.sync_copy(nd.at[pl.ds(w * 16, 16)], ndb)
    pltpu.sync_copy(zbig.at[pl.ds(s * STRIDE, STRIDE)],
                    acc_cnt.at[pl.ds(s * STRIDE, STRIDE)])

    for j in range(6):
        pltpu.async_copy(table.at[ndb.at[j]], rb0, sem).wait()
        pltpu.async_copy(table.at[ndb.at[8 + j]], oh128, sem).wait()

        @pl.loop(0, CH, unroll=8)
        def _add(r):
            for q in range(8):
                rb0[r, pl.ds(16 * q, 16)] = (rb0[r, pl.ds(16 * q, 16)]
                                             + oh128[r, pl.ds(16 * q, 16)])

        pltpu.sync_copy(rb0, h0_out.at[pl.ds(w * NPT + j * CH, CH)])

    @pl.loop(0, CH)
    def _z(r):
        for q in range(8):
            oh128[r, pl.ds(16 * q, 16)] = jnp.zeros((16,), f32)

    plsc.subcore_barrier()

    for p in range(NPH):
        pltpu.sync_copy(ed.at[pl.ds(w * NCHUNK + p * CPP, CPP)], edb)
        base = w * EPT + p * CPP * CH
        pltpu.async_copy(ohp.at[pl.ds(base, CH)], oh0, semo0)

        @pl.loop(0, CPP // 2)
        def _hist(i):
            k = 2 * i
            pltpu.async_copy(ohp.at[pl.ds(base + (k + 1) * CH, CH)], oh1, semo1)
            pltpu.make_async_copy(ohp.at[pl.ds(base, CH)], oh0, semo0).wait()

            @pl.loop(0, CH, unroll=8)
            def _exp0(r):
                oh128[r, pl.ds(0, 16)] = oh0[r, :]

            pltpu.sync_copy(oh128, acc_cnt.at[edb.at[k]], add=True)

            @pl.when(k + 2 < CPP)
            def _():
                pltpu.async_copy(ohp.at[pl.ds(base + (k + 2) * CH, CH)], oh0,
                                 semo0)

            pltpu.make_async_copy(ohp.at[pl.ds(base, CH)], oh1, semo1).wait()

            @pl.loop(0, CH, unroll=8)
            def _exp1(r):
                oh128[r, pl.ds(0, 16)] = oh1[r, :]

            pltpu.sync_copy(oh128, acc_cnt.at[edb.at[k + 1]], add=True)

    plsc.subcore_barrier()
    pltpu.sync_copy(acc_cnt.at[pl.ds(s * STRIDE, STRIDE)],
                    cnt_out.at[c, pl.ds(s * STRIDE, STRIDE)])


NPH = 4
CPP = NCHUNK // NPH

KC0, KC1 = 224, 96
KTOT = KC0 + KC1
CPPC = 32


def _sc_scatter(h, sd, zbig, out, acc, sdb, rb0, rb1, sem0, sem1):
    c = lax.axis_index("c")
    s = lax.axis_index("s")
    with jax.named_scope("zero_acc"):
        pltpu.sync_copy(zbig.at[pl.ds(s * STRIDE, STRIDE)],
                        acc.at[pl.ds(s * STRIDE, STRIDE)])
        plsc.subcore_barrier()

    base_row = s * (2 * KTOT) + c * (2 * KC0)
    nph = (KC0 // CPPC) - ((KC0 - KC1) // CPPC) * c

    @pl.loop(0, nph)
    def _phase(p):
        with jax.named_scope("gscat_phase"):
            pltpu.sync_copy(sd.at[pl.ds(base_row + p * 2 * CPPC, 2 * CPPC)],
                            sdb)
            pltpu.async_copy(h.at[sdb.at[0]], rb0, sem0)

            @pl.loop(0, CPPC // 2)
            def _pair(i):
                k = 2 * i
                pltpu.async_copy(h.at[sdb.at[2 * k + 2]], rb1, sem1)
                pltpu.make_async_copy(h.at[sdb.at[2 * k]], rb0, sem0).wait()
                pltpu.sync_copy(rb0, acc.at[sdb.at[2 * k + 1]], add=True)

                @pl.when(k + 2 < CPPC)
                def _():
                    pltpu.async_copy(h.at[sdb.at[2 * k + 4]], rb0, sem0)

                pltpu.make_async_copy(h.at[sdb.at[2 * k + 2]], rb1,
                                      sem1).wait()
                pltpu.sync_copy(rb1, acc.at[sdb.at[2 * k + 3]], add=True)

    plsc.subcore_barrier()
    with jax.named_scope("writeout"):
        pltpu.sync_copy(acc.at[pl.ds(s * STRIDE, STRIDE)],
                        out.at[c, pl.ds(s * STRIDE, STRIDE)])


def _scatter_call(h, sd, zbig):
    return pl.kernel(
        _sc_scatter,
        out_type=jax.ShapeDtypeStruct((NC, NA, D), f32),
        mesh=_mesh(),
        scratch_types=[
            pltpu.VMEM_SHARED((NA, D), f32),
            pltpu.VMEM((2 * CPPC, CH), i32),
            pltpu.VMEM((CH, D), f32),
            pltpu.VMEM((CH, D), f32),
            pltpu.SemaphoreType.DMA,
            pltpu.SemaphoreType.DMA,
        ],
    )(h, sd, zbig)


def _tc_update(S_ref, h_ref, cnt_ref, Ew_ref, W_ref, b_ref, se_ref, out_ref,
               *, relu):
    cc = cnt_ref[0] + cnt_ref[1]
    aggr = (S_ref[0] + S_ref[1] + h_ref[...] + se_ref[...]
            + jnp.dot(cc, Ew_ref[...], preferred_element_type=f32))
    o = jnp.dot(aggr, W_ref[...], preferred_element_type=f32) + b_ref[...]
    if relu:
        o = jnp.maximum(o, 0.0)
    out_ref[...] = o


def _update_call(S, h, cnt, Ew_l, W_l, b_l, se_l, relu):
    R = 1000
    return pl.pallas_call(
        functools.partial(_tc_update, relu=relu),
        grid=(N // R,),
        in_specs=[
            pl.BlockSpec((NC, R, D), lambda i: (0, i, 0)),
            pl.BlockSpec((R, D), lambda i: (i, 0)),
            pl.BlockSpec((NC, R, D), lambda i: (0, i, 0)),
            pl.BlockSpec((D, D), lambda i: (0, 0)),
            pl.BlockSpec((D, D), lambda i: (0, 0)),
            pl.BlockSpec((1, D), lambda i: (0, 0)),
            pl.BlockSpec((1, D), lambda i: (0, 0)),
        ],
        out_specs=pl.BlockSpec((R, D), lambda i: (i, 0)),
        out_shape=jax.ShapeDtypeStruct((N, D), f32),
    )(S, h, cnt, Ew_l, W_l, b_l.reshape(1, D), se_l.reshape(1, D))


def kernel(x, edge_index, edge_attr, x_emb1, x_emb2, mlp_W, mlp_b, ee1, ee2,
           bn_g, bn_b):
    x = x.astype(i32)
    edge_index = edge_index.astype(i32)
    edge_attr = edge_attr.astype(i32)

    x0 = jnp.clip(x[:, 0], 0, 119)
    x1 = jnp.clip(x[:, 1], 0, 3) + 120
    src = jnp.clip(edge_index[0], 0, N - 1)
    dst = jnp.clip(edge_index[1], 0, N - 1)
    ea0 = jnp.clip(edge_attr[:, 0], 0, 6)
    ea1 = jnp.clip(edge_attr[:, 1], 0, 3) + 8

    x0p = jnp.concatenate([x0, jnp.zeros(NP - N, i32)])
    x1p = jnp.concatenate([x1, jnp.full(NP - N, 120, i32)])
    nd = jnp.concatenate([x0p.reshape(NT, 6, CH),
                          jnp.zeros((NT, 2, CH), i32),
                          x1p.reshape(NT, 6, CH),
                          jnp.zeros((NT, 2, CH), i32)],
                         axis=1).reshape(NT * 16, CH)

    pe = EP - E
    ea0p = jnp.concatenate([ea0, jnp.zeros(pe, i32)])
    ea1p = jnp.concatenate([ea1, jnp.full(pe, 8, i32)])
    dstp = jnp.concatenate([dst, jnp.full(pe, N, i32)])
    srcp = jnp.concatenate([src, jnp.zeros(pe, i32)])
    ed = dstp.reshape(NT * NCHUNK, CH)
    sd = jnp.stack([srcp.reshape(NS, KTOT, CH),
                    dstp.reshape(NS, KTOT, CH)], axis=2).reshape(-1, CH)
    cols = jnp.arange(16, dtype=i32)[None, :]
    ohp = ((ea0p[:, None] == cols) | (ea1p[:, None] == cols)).astype(f32)

    tbl = jnp.zeros((128, D), f32).at[0:120].set(x_emb1).at[120:124].set(x_emb2)
    zbig = jnp.zeros((NA, D), f32)

    scl = bn_g / jnp.sqrt(1.0 + 1e-5)
    Wp = jnp.transpose(mlp_W, (0, 2, 1)) * scl[:, None, :]
    bias = mlp_b * scl + bn_b
    Ew = jnp.zeros((3, D, D), f32).at[:, 0:7].set(ee1).at[:, 8:12].set(ee2)
    se = ee1[:, 4] + ee2[:, 0]

    h0, cnt = pl.kernel(
        _sc_init,
        out_type=(jax.ShapeDtypeStruct((NP, D), f32),
                  jax.ShapeDtypeStruct((NC, NA, D), f32)),
        mesh=_mesh(),
        scratch_types=[
            pltpu.VMEM_SHARED((NA, D), f32),
            pltpu.VMEM((16, CH), i32),
            pltpu.VMEM((CPP, CH), i32),
            pltpu.VMEM((CH, D), f32),
            pltpu.VMEM((CH, D), f32),
            pltpu.VMEM((CH, 16), f32),
            pltpu.VMEM((CH, 16), f32),
            pltpu.SemaphoreType.DMA,
            pltpu.SemaphoreType.DMA,
            pltpu.SemaphoreType.DMA,
        ],
    )(nd, ed, ohp, tbl, zbig)

    h = h0
    for l in range(3):
        S = _scatter_call(h, sd, zbig)
        h = _update_call(S, h, cnt, Ew[l], Wp[l], bias[l], se[l], relu=(l < 2))
    return h


# speedup vs baseline: 1.5775x; 1.5775x over previous
import functools

import jax
import jax.numpy as jnp
from jax import lax
from jax.experimental import pallas as pl
from jax.experimental.pallas import tpu as pltpu
from jax.experimental.pallas import tpu_sc as plsc

N = 10000
E = 320000
D = 128
NC, NS = 2, 16
NT = NC * NS
CH = 64
EPT = 10240
EP = EPT * NT
NCHUNK = EPT // CH
NPT = 384
NP = NPT * NT
NA = 10240
STRIDE = NA // NS

f32 = jnp.float32
i32 = jnp.int32


def _mesh():
    return plsc.VectorSubcoreMesh(
        core_axis_name="c", subcore_axis_name="s", num_cores=NC, num_subcores=NS
    )


def _sc_init(nd, ed, ohp, table, zbig, h0_out, cnt_out, acc_cnt, ndb, edb,
             rb0, oh128, oh0, oh1, sem, semo0, semo1):
    c = lax.axis_index("c")
    s = lax.axis_index("s")
    w = s * NC + c
    pltpu.sync_copy(nd.at[pl.ds(w * 16, 16)], ndb)
    pltpu.sync_copy(zbig.at[pl.ds(s * STRIDE, STRIDE)],
                    acc_cnt.at[pl.ds(s * STRIDE, STRIDE)])

    for j in range(6):
        pltpu.async_copy(table.at[ndb.at[j]], rb0, sem).wait()
        pltpu.async_copy(table.at[ndb.at[8 + j]], oh128, sem).wait()

        @pl.loop(0, CH, unroll=8)
        def _add(r):
            for q in range(8):
                rb0[r, pl.ds(16 * q, 16)] = (rb0[r, pl.ds(16 * q, 16)]
                                             + oh128[r, pl.ds(16 * q, 16)])

        pltpu.sync_copy(rb0, h0_out.at[pl.ds(w * NPT + j * CH, CH)])

    @pl.loop(0, CH)
    def _z(r):
        for q in range(8):
            oh128[r, pl.ds(16 * q, 16)] = jnp.zeros((16,), f32)

    plsc.subcore_barrier()

    for p in range(NPH):
        pltpu.sync_copy(ed.at[pl.ds(w * NCHUNK + p * CPP, CPP)], edb)
        base = w * EPT + p * CPP * CH
        pltpu.async_copy(ohp.at[pl.ds(base, CH)], oh0, semo0)

        @pl.loop(0, CPP // 2)
        def _hist(i):
            k = 2 * i
            pltpu.async_copy(ohp.at[pl.ds(base + (k + 1) * CH, CH)], oh1, semo1)
            pltpu.make_async_copy(ohp.at[pl.ds(base, CH)], oh0, semo0).wait()

            @pl.loop(0, CH, unroll=8)
            def _exp0(r):
                oh128[r, pl.ds(0, 16)] = oh0[r, :]

            pltpu.sync_copy(oh128, acc_cnt.at[edb.at[k]], add=True)

            @pl.when(k + 2 < CPP)
            def _():
                pltpu.async_copy(ohp.at[pl.ds(base + (k + 2) * CH, CH)], oh0,
                                 semo0)

            pltpu.make_async_copy(ohp.at[pl.ds(base, CH)], oh1, semo1).wait()

            @pl.loop(0, CH, unroll=8)
            def _exp1(r):
                oh128[r, pl.ds(0, 16)] = oh1[r, :]

            pltpu.sync_copy(oh128, acc_cnt.at[edb.at[k + 1]], add=True)

    plsc.subcore_barrier()
    pltpu.sync_copy(acc_cnt.at[pl.ds(s * STRIDE, STRIDE)],
                    cnt_out.at[c, pl.ds(s * STRIDE, STRIDE)])


NPH = 4
CPP = NCHUNK // NPH

KC0, KC1 = 224, 96
KTOT = KC0 + KC1
CPPC = 32


def _sc_scatter(h, sd, zbig, out, acc, sdb, rb0, rb1, sem0, sem1):
    c = lax.axis_index("c")
    s = lax.axis_index("s")
    with jax.named_scope("zero_acc"):
        pltpu.sync_copy(zbig.at[pl.ds(s * STRIDE, STRIDE)],
                        acc.at[pl.ds(s * STRIDE, STRIDE)])
        plsc.subcore_barrier()

    base_row = s * (2 * KTOT) + c * (2 * KC0)
    nph = (KC0 // CPPC) - ((KC0 - KC1) // CPPC) * c

    @pl.loop(0, nph)
    def _phase(p):
        with jax.named_scope("gscat_phase"):
            pltpu.sync_copy(sd.at[pl.ds(base_row + p * 2 * CPPC, 2 * CPPC)],
                            sdb)
            pltpu.async_copy(h.at[sdb.at[0]], rb0, sem0)

            @pl.loop(0, CPPC // 2)
            def _pair(i):
                k = 2 * i
                pltpu.async_copy(h.at[sdb.at[2 * k + 2]], rb1, sem1)
                pltpu.make_async_copy(h.at[sdb.at[2 * k]], rb0, sem0).wait()
                pltpu.sync_copy(rb0, acc.at[sdb.at[2 * k + 1]], add=True)

                @pl.when(k + 2 < CPPC)
                def _():
                    pltpu.async_copy(h.at[sdb.at[2 * k + 4]], rb0, sem0)

                pltpu.make_async_copy(h.at[sdb.at[2 * k + 2]], rb1,
                                      sem1).wait()
                pltpu.sync_copy(rb1, acc.at[sdb.at[2 * k + 3]], add=True)

    plsc.subcore_barrier()
    with jax.named_scope("writeout"):
        pltpu.sync_copy(acc.at[pl.ds(s * STRIDE, STRIDE)],
                        out.at[c, pl.ds(s * STRIDE, STRIDE)])


def _scatter_call(h, sd, zbig):
    return pl.kernel(
        _sc_scatter,
        out_type=jax.ShapeDtypeStruct((NC, NA, D), f32),
        mesh=_mesh(),
        scratch_types=[
            pltpu.VMEM_SHARED((NA, D), f32),
            pltpu.VMEM((2 * CPPC, CH), i32),
            pltpu.VMEM((CH, D), f32),
            pltpu.VMEM((CH, D), f32),
            pltpu.SemaphoreType.DMA,
            pltpu.SemaphoreType.DMA,
        ],
    )(h, sd, zbig)


def _tc_update(S_ref, h_ref, cnt_ref, Ew_ref, W_ref, b_ref, se_ref, out_ref,
               *, relu):
    cc = cnt_ref[0] + cnt_ref[1]
    aggr = (S_ref[0] + S_ref[1] + h_ref[...] + se_ref[...]
            + jnp.dot(cc, Ew_ref[...], preferred_element_type=f32))
    o = jnp.dot(aggr, W_ref[...], preferred_element_type=f32) + b_ref[...]
    if relu:
        o = jnp.maximum(o, 0.0)
    out_ref[...] = o


def _update_call(S, h, cnt, Ew_l, W_l, b_l, se_l, relu):
    R = 1000
    return pl.pallas_call(
        functools.partial(_tc_update, relu=relu),
        grid=(N // R,),
        in_specs=[
            pl.BlockSpec((NC, R, D), lambda i: (0, i, 0)),
            pl.BlockSpec((R, D), lambda i: (i, 0)),
            pl.BlockSpec((NC, R, D), lambda i: (0, i, 0)),
            pl.BlockSpec((D, D), lambda i: (0, 0)),
            pl.BlockSpec((D, D), lambda i: (0, 0)),
            pl.BlockSpec((1, D), lambda i: (0, 0)),
            pl.BlockSpec((1, D), lambda i: (0, 0)),
        ],
        out_specs=pl.BlockSpec((R, D), lambda i: (i, 0)),
        out_shape=jax.ShapeDtypeStruct((N, D), f32),
    )(S, h, cnt, Ew_l, W_l, b_l.reshape(1, D), se_l.reshape(1, D))


def kernel(x, edge_index, edge_attr, x_emb1, x_emb2, mlp_W, mlp_b, ee1, ee2,
           bn_g, bn_b):
    x = x.astype(i32)
    edge_index = edge_index.astype(i32)
    edge_attr = edge_attr.astype(i32)

    x0 = jnp.clip(x[:, 0], 0, 119)
    x1 = jnp.clip(x[:, 1], 0, 3) + 120
    src = jnp.clip(edge_index[0], 0, N - 1)
    dst = jnp.clip(edge_index[1], 0, N - 1)
    ea0 = jnp.clip(edge_attr[:, 0], 0, 6)
    ea1 = jnp.clip(edge_attr[:, 1], 0, 3) + 8

    x0p = jnp.concatenate([x0, jnp.zeros(NP - N, i32)])
    x1p = jnp.concatenate([x1, jnp.full(NP - N, 120, i32)])
    nd = jnp.concatenate([x0p.reshape(NT, 6, CH),
                          jnp.zeros((NT, 2, CH), i32),
                          x1p.reshape(NT, 6, CH),
                          jnp.zeros((NT, 2, CH), i32)],
                         axis=1).reshape(NT * 16, CH)

    pe = EP - E
    pad = jnp.arange(pe, dtype=i32)
    ea0p = jnp.concatenate([ea0, jnp.zeros(pe, i32)])
    ea1p = jnp.concatenate([ea1, jnp.full(pe, 8, i32)])
    dstp = jnp.concatenate([dst, N + pad % (NA - N)])
    srcp = jnp.concatenate([src, pad % N])
    ed = dstp.reshape(NT * NCHUNK, CH)
    sd = jnp.stack([srcp.reshape(NS, KTOT, CH),
                    dstp.reshape(NS, KTOT, CH)], axis=2).reshape(-1, CH)
    cols = jnp.arange(16, dtype=i32)[None, :]
    ohp = ((ea0p[:, None] == cols) | (ea1p[:, None] == cols)).astype(f32)

    tbl = jnp.zeros((128, D), f32).at[0:120].set(x_emb1).at[120:124].set(x_emb2)
    zbig = jnp.zeros((NA, D), f32)

    scl = bn_g / jnp.sqrt(1.0 + 1e-5)
    Wp = jnp.transpose(mlp_W, (0, 2, 1)) * scl[:, None, :]
    bias = mlp_b * scl + bn_b
    Ew = jnp.zeros((3, D, D), f32).at[:, 0:7].set(ee1).at[:, 8:12].set(ee2)
    se = ee1[:, 4] + ee2[:, 0]

    h0, cnt = pl.kernel(
        _sc_init,
        out_type=(jax.ShapeDtypeStruct((NP, D), f32),
                  jax.ShapeDtypeStruct((NC, NA, D), f32)),
        mesh=_mesh(),
        scratch_types=[
            pltpu.VMEM_SHARED((NA, D), f32),
            pltpu.VMEM((16, CH), i32),
            pltpu.VMEM((CPP, CH), i32),
            pltpu.VMEM((CH, D), f32),
            pltpu.VMEM((CH, D), f32),
            pltpu.VMEM((CH, 16), f32),
            pltpu.VMEM((CH, 16), f32),
            pltpu.SemaphoreType.DMA,
            pltpu.SemaphoreType.DMA,
            pltpu.SemaphoreType.DMA,
        ],
    )(nd, ed, ohp, tbl, zbig)

    h = h0
    for l in range(3):
        S = _scatter_call(h, sd, zbig)
        h = _update_call(S, h, cnt, Ew[l], Wp[l], bias[l], se[l], relu=(l < 2))
    return h
